# baseline (speedup 1.0000x reference)
import jax
import jax.numpy as jnp
from jax import lax
from jax.experimental import pallas as pl
from jax.experimental.pallas import tpu as pltpu

_EPS = 1e-5
_BLK = 2048


def _dot(a, b, dims):
    return lax.dot_general(a, b, (dims, ((), ())),
                           preferred_element_type=jnp.float32)


def _body(o_ref, xT_hbm, W1_ref, b1_ref, gamma_ref, beta_ref, W2_ref, b2_ref,
          outT_ref, xv, G_ref, sums_ref, W1as_ref, c2_ref, sems):
    d, n = xv.shape
    nb = o_ref.shape[1]
    nblk = n // _BLK
    i = pl.program_id(0)
    nf = jnp.float32(n)

    of_row = o_ref[...].astype(jnp.float32)
    eye_b = (lax.broadcasted_iota(jnp.int32, (nb, nb), 0)
             == lax.broadcasted_iota(jnp.int32, (nb, nb), 1)
             ).astype(jnp.float32)
    o_col = _dot(eye_b, of_row, ((1,), (1,)))

    def mask_for(base):
        rf = (lax.broadcasted_iota(jnp.int32, (nb, _BLK), 1)
              + base).astype(jnp.float32)
        ltf = (rf < o_col).astype(jnp.float32)
        lt_prev = jnp.concatenate(
            [jnp.zeros((1, _BLK), jnp.float32), ltf[:-1, :]], axis=0)
        return ltf - lt_prev

    def copy_blk(base, sem):
        return pltpu.make_async_copy(
            xT_hbm.at[:, pl.ds(base, _BLK)], xv.at[:, pl.ds(base, _BLK)], sem)

    @pl.when(i == 0)
    def _():
        G_ref[...] = jnp.zeros_like(G_ref)
        sums_ref[...] = jnp.zeros_like(sums_ref)
        copy_blk(0, sems.at[0]).start()

    @pl.when(i + 1 < nblk)
    def _():
        copy_blk((i + 1) * _BLK, sems.at[(i + 1) % 2]).start()

    @pl.when(i < nblk)
    def _():
        base = i * _BLK
        copy_blk(base, sems.at[i % 2]).wait()
        xb = xv[:, pl.ds(base, _BLK)]
        G_ref[...] += _dot(xb, xb, ((1,), (1,)))
        sums_ref[...] += _dot(xb, mask_for(base), ((1,), (1,)))

    @pl.when(i == nblk)
    def _():
        prev_col = jnp.concatenate(
            [jnp.zeros((1, 1), jnp.float32), o_col[:-1, :]], axis=0)
        cnt_col = o_col - prev_col
        cnt_row = of_row - jnp.concatenate(
            [jnp.zeros((1, 1), jnp.float32), of_row[:, :-1]], axis=1)
        eye_d = (lax.broadcasted_iota(jnp.int32, (d, d), 0)
                 == lax.broadcasted_iota(jnp.int32, (d, d), 1)
                 ).astype(jnp.float32)
        vecs = jnp.concatenate(
            [b1_ref[...], gamma_ref[...], beta_ref[...], b2_ref[...]], axis=0)
        vecs_col = _dot(eye_d, vecs, ((1,), (1,)))
        b1_col = vecs_col[:, 0:1]
        gamma_col = vecs_col[:, 1:2]
        beta_col = vecs_col[:, 2:3]
        b2_col = vecs_col[:, 3:4]

        sums_T = sums_ref[...]
        m_T = sums_T * (1.0 / cnt_row)
        h_T = jnp.maximum(
            _dot(W2_ref[...], m_T, ((1,), (0,))) + b2_col, 0.0)
        W1 = W1_ref[...]
        W1a = W1[:, :d]
        W1b = W1[:, d:]
        c_T = _dot(W1b, h_T, ((1,), (0,))) + b1_col

        seg_t_T = _dot(W1a, sums_T, ((1,), (0,)))
        sum_y = (jnp.sum(seg_t_T, axis=1, keepdims=True)
                 + jnp.sum(c_T * cnt_row, axis=1, keepdims=True))
        W1aG = _dot(W1a, G_ref[...], ((1,), (0,)))
        sumsq_t = jnp.sum(W1a * W1aG, axis=1, keepdims=True)
        sumsq_y = (sumsq_t
                   + 2.0 * jnp.sum(c_T * seg_t_T, axis=1, keepdims=True)
                   + jnp.sum(c_T * c_T * cnt_row, axis=1, keepdims=True))
        mean = sum_y / nf
        var = sumsq_y / nf - mean * mean
        a_col = gamma_col * lax.rsqrt(var + _EPS)
        bsh_col = beta_col - mean * a_col
        W1as_ref[...] = W1a * a_col
        c2_ref[...] = c_T * a_col + bsh_col

    @pl.when(i >= nblk)
    def _():
        base = (i - nblk) * _BLK
        xb = xv[:, pl.ds(base, _BLK)]
        outT_ref[...] = jnp.maximum(
            _dot(W1as_ref[...], xb, ((1,), (0,)))
            + _dot(c2_ref[...], mask_for(base), ((1,), (0,))), 0.0)


def kernel(p, x, o, W1, b1, gamma, beta, W2, b2):
    del p
    n, d = x.shape
    nb = o.shape[0]
    nblk = n // _BLK
    xT = jnp.swapaxes(x, 0, 1)
    outT = pl.pallas_call(
        _body,
        grid=(2 * nblk,),
        in_specs=[
            pl.BlockSpec((1, nb), lambda i: (0, 0)),
            pl.BlockSpec(memory_space=pl.ANY),
            pl.BlockSpec((d, 2 * d), lambda i: (0, 0)),
            pl.BlockSpec((1, d), lambda i: (0, 0)),
            pl.BlockSpec((1, d), lambda i: (0, 0)),
            pl.BlockSpec((1, d), lambda i: (0, 0)),
            pl.BlockSpec((d, d), lambda i: (0, 0)),
            pl.BlockSpec((1, d), lambda i: (0, 0)),
        ],
        out_specs=pl.BlockSpec(
            (d, _BLK), lambda i: (0, jnp.maximum(i - n // _BLK, 0))),
        out_shape=jax.ShapeDtypeStruct((d, n), x.dtype),
        scratch_shapes=[
            pltpu.VMEM((d, n), jnp.float32),
            pltpu.VMEM((d, d), jnp.float32),
            pltpu.VMEM((d, nb), jnp.float32),
            pltpu.VMEM((d, d), jnp.float32),
            pltpu.VMEM((d, nb), jnp.float32),
            pltpu.SemaphoreType.DMA((2,)),
        ],
    )(o.reshape(1, nb), xT, W1, b1.reshape(1, d), gamma.reshape(1, d),
      beta.reshape(1, d), W2, b2.reshape(1, d))
    return jnp.swapaxes(outT, 0, 1)


# speedup vs baseline: 1.4814x; 1.4814x over previous
import jax
import jax.numpy as jnp
from jax import lax
from jax.experimental import pallas as pl
from jax.experimental.pallas import tpu as pltpu

_EPS = 1e-5
_BLK = 4096


def _dot(a, b, dims):
    return lax.dot_general(a, b, (dims, ((), ())),
                           preferred_element_type=jnp.float32)


def _body(o_ref, xT_hbm, W1_ref, b1_ref, gamma_ref, beta_ref, W2_ref, b2_ref,
          outT_ref, xv, G_ref, sums_ref, W1as_ref, c2_ref, sems):
    _, d, _ = xv.shape
    n = xv.shape[0] * _BLK
    nb = o_ref.shape[1]
    nblk = n // _BLK
    i = pl.program_id(0)
    nf = jnp.float32(n)

    of_row = o_ref[...].astype(jnp.float32)
    eye_b = (lax.broadcasted_iota(jnp.int32, (nb, nb), 0)
             == lax.broadcasted_iota(jnp.int32, (nb, nb), 1)
             ).astype(jnp.float32)
    o_col = _dot(eye_b, of_row, ((1,), (1,)))

    def mask_for(base):
        rf = (lax.broadcasted_iota(jnp.int32, (nb, _BLK), 1)
              + base).astype(jnp.float32)
        ltf = (rf < o_col).astype(jnp.float32)
        lt_prev = jnp.concatenate(
            [jnp.zeros((1, _BLK), jnp.float32), ltf[:-1, :]], axis=0)
        return ltf - lt_prev

    def copy_blk(j, sem):
        return pltpu.make_async_copy(
            xT_hbm.at[:, pl.ds(j * _BLK, _BLK)], xv.at[j], sem)

    @pl.when(i == 0)
    def _():
        G_ref[...] = jnp.zeros_like(G_ref)
        sums_ref[...] = jnp.zeros_like(sums_ref)
        copy_blk(0, sems.at[0]).start()

    @pl.when(i + 1 < nblk)
    def _():
        copy_blk(i + 1, sems.at[(i + 1) % 2]).start()

    @pl.when(i < nblk)
    def _():
        copy_blk(i, sems.at[i % 2]).wait()
        xb = xv[i]
        G_ref[...] += _dot(xb, xb, ((1,), (1,)))
        sums_ref[...] += _dot(xb, mask_for(i * _BLK), ((1,), (1,)))

    @pl.when(i == nblk)
    def _():
        prev_col = jnp.concatenate(
            [jnp.zeros((1, 1), jnp.float32), o_col[:-1, :]], axis=0)
        cnt_col = o_col - prev_col
        cnt_row = of_row - jnp.concatenate(
            [jnp.zeros((1, 1), jnp.float32), of_row[:, :-1]], axis=1)
        eye_d = (lax.broadcasted_iota(jnp.int32, (d, d), 0)
                 == lax.broadcasted_iota(jnp.int32, (d, d), 1)
                 ).astype(jnp.float32)
        vecs = jnp.concatenate(
            [b1_ref[...], gamma_ref[...], beta_ref[...], b2_ref[...]], axis=0)
        vecs_col = _dot(eye_d, vecs, ((1,), (1,)))
        b1_col = vecs_col[:, 0:1]
        gamma_col = vecs_col[:, 1:2]
        beta_col = vecs_col[:, 2:3]
        b2_col = vecs_col[:, 3:4]

        sums_T = sums_ref[...]
        m_T = sums_T * (1.0 / cnt_row)
        h_T = jnp.maximum(
            _dot(W2_ref[...], m_T, ((1,), (0,))) + b2_col, 0.0)
        W1 = W1_ref[...]
        W1a = W1[:, :d]
        W1b = W1[:, d:]
        c_T = _dot(W1b, h_T, ((1,), (0,))) + b1_col

        seg_t_T = _dot(W1a, sums_T, ((1,), (0,)))
        sum_y = (jnp.sum(seg_t_T, axis=1, keepdims=True)
                 + jnp.sum(c_T * cnt_row, axis=1, keepdims=True))
        W1aG = _dot(W1a, G_ref[...], ((1,), (0,)))
        sumsq_t = jnp.sum(W1a * W1aG, axis=1, keepdims=True)
        sumsq_y = (sumsq_t
                   + 2.0 * jnp.sum(c_T * seg_t_T, axis=1, keepdims=True)
                   + jnp.sum(c_T * c_T * cnt_row, axis=1, keepdims=True))
        mean = sum_y / nf
        var = sumsq_y / nf - mean * mean
        a_col = gamma_col * lax.rsqrt(var + _EPS)
        bsh_col = beta_col - mean * a_col
        W1as_ref[...] = W1a * a_col
        c2_ref[...] = c_T * a_col + bsh_col

    @pl.when(i >= nblk)
    def _():
        base = (i - nblk) * _BLK
        xb = xv[i - nblk]
        outT_ref[...] = jnp.maximum(
            _dot(W1as_ref[...], xb, ((1,), (0,)))
            + _dot(c2_ref[...], mask_for(base), ((1,), (0,))), 0.0)


def kernel(p, x, o, W1, b1, gamma, beta, W2, b2):
    del p
    n, d = x.shape
    nb = o.shape[0]
    nblk = n // _BLK
    xT = jnp.swapaxes(x, 0, 1)
    outT = pl.pallas_call(
        _body,
        grid=(2 * nblk,),
        in_specs=[
            pl.BlockSpec((1, nb), lambda i: (0, 0)),
            pl.BlockSpec(memory_space=pl.ANY),
            pl.BlockSpec((d, 2 * d), lambda i: (0, 0)),
            pl.BlockSpec((1, d), lambda i: (0, 0)),
            pl.BlockSpec((1, d), lambda i: (0, 0)),
            pl.BlockSpec((1, d), lambda i: (0, 0)),
            pl.BlockSpec((d, d), lambda i: (0, 0)),
            pl.BlockSpec((1, d), lambda i: (0, 0)),
        ],
        out_specs=pl.BlockSpec(
            (d, _BLK), lambda i: (0, jnp.maximum(i - n // _BLK, 0))),
        out_shape=jax.ShapeDtypeStruct((d, n), x.dtype),
        scratch_shapes=[
            pltpu.VMEM((n // _BLK, d, _BLK), jnp.float32),
            pltpu.VMEM((d, d), jnp.float32),
            pltpu.VMEM((d, nb), jnp.float32),
            pltpu.VMEM((d, d), jnp.float32),
            pltpu.VMEM((d, nb), jnp.float32),
            pltpu.SemaphoreType.DMA((2,)),
        ],
    )(o.reshape(1, nb), xT, W1, b1.reshape(1, d), gamma.reshape(1, d),
      beta.reshape(1, d), W2, b2.reshape(1, d))
    return jnp.swapaxes(outT, 0, 1)


# speedup vs baseline: 1.8966x; 1.2802x over previous
import jax
import jax.numpy as jnp
from jax import lax
from jax.experimental import pallas as pl
from jax.experimental.pallas import tpu as pltpu

_EPS = 1e-5
_BLK = 8192


def _dot(a, b, dims):
    return lax.dot_general(a, b, (dims, ((), ())),
                           preferred_element_type=jnp.float32)


def _body(o_ref, xT_hbm, W1_ref, b1_ref, gamma_ref, beta_ref, W2_ref, b2_ref,
          outT_ref, xv, G_ref, sums_ref, W1as_ref, c2_ref, sems):
    _, d, _ = xv.shape
    n = xv.shape[0] * _BLK
    nb = o_ref.shape[1]
    nblk = n // _BLK
    i = pl.program_id(0)
    nf = jnp.float32(n)

    of_row = o_ref[...].astype(jnp.float32)
    eye_b = (lax.broadcasted_iota(jnp.int32, (nb, nb), 0)
             == lax.broadcasted_iota(jnp.int32, (nb, nb), 1)
             ).astype(jnp.float32)
    o_col = _dot(eye_b, of_row, ((1,), (1,)))

    def mask_for(base):
        rf = (lax.broadcasted_iota(jnp.int32, (nb, _BLK), 1)
              + base).astype(jnp.float32)
        ltf = (rf < o_col).astype(jnp.float32)
        lt_prev = jnp.concatenate(
            [jnp.zeros((1, _BLK), jnp.float32), ltf[:-1, :]], axis=0)
        return ltf - lt_prev

    def copy_blk(j, sem):
        return pltpu.make_async_copy(
            xT_hbm.at[:, pl.ds(j * _BLK, _BLK)], xv.at[j], sem)

    @pl.when(i == 0)
    def _():
        G_ref[...] = jnp.zeros_like(G_ref)
        sums_ref[...] = jnp.zeros_like(sums_ref)
        copy_blk(0, sems.at[0]).start()

    @pl.when(i + 1 < nblk)
    def _():
        copy_blk(i + 1, sems.at[(i + 1) % 2]).start()

    @pl.when(i < nblk)
    def _():
        copy_blk(i, sems.at[i % 2]).wait()
        xb = xv[i]
        G_ref[...] += _dot(xb, xb, ((1,), (1,)))
        sums_ref[...] += _dot(xb, mask_for(i * _BLK), ((1,), (1,)))

    @pl.when(i == nblk)
    def _():
        prev_col = jnp.concatenate(
            [jnp.zeros((1, 1), jnp.float32), o_col[:-1, :]], axis=0)
        cnt_col = o_col - prev_col
        cnt_row = of_row - jnp.concatenate(
            [jnp.zeros((1, 1), jnp.float32), of_row[:, :-1]], axis=1)
        eye_d = (lax.broadcasted_iota(jnp.int32, (d, d), 0)
                 == lax.broadcasted_iota(jnp.int32, (d, d), 1)
                 ).astype(jnp.float32)
        vecs = jnp.concatenate(
            [b1_ref[...], gamma_ref[...], beta_ref[...], b2_ref[...]], axis=0)
        vecs_col = _dot(eye_d, vecs, ((1,), (1,)))
        b1_col = vecs_col[:, 0:1]
        gamma_col = vecs_col[:, 1:2]
        beta_col = vecs_col[:, 2:3]
        b2_col = vecs_col[:, 3:4]

        sums_T = sums_ref[...]
        m_T = sums_T * (1.0 / cnt_row)
        h_T = jnp.maximum(
            _dot(W2_ref[...], m_T, ((1,), (0,))) + b2_col, 0.0)
        W1 = W1_ref[...]
        W1a = W1[:, :d]
        W1b = W1[:, d:]
        c_T = _dot(W1b, h_T, ((1,), (0,))) + b1_col

        seg_t_T = _dot(W1a, sums_T, ((1,), (0,)))
        sum_y = (jnp.sum(seg_t_T, axis=1, keepdims=True)
                 + jnp.sum(c_T * cnt_row, axis=1, keepdims=True))
        W1aG = _dot(W1a, G_ref[...], ((1,), (0,)))
        sumsq_t = jnp.sum(W1a * W1aG, axis=1, keepdims=True)
        sumsq_y = (sumsq_t
                   + 2.0 * jnp.sum(c_T * seg_t_T, axis=1, keepdims=True)
                   + jnp.sum(c_T * c_T * cnt_row, axis=1, keepdims=True))
        mean = sum_y / nf
        var = sumsq_y / nf - mean * mean
        a_col = gamma_col * lax.rsqrt(var + _EPS)
        bsh_col = beta_col - mean * a_col
        W1as_ref[...] = W1a * a_col
        c2_ref[...] = c_T * a_col + bsh_col

    @pl.when(i >= nblk)
    def _():
        base = (i - nblk) * _BLK
        xb = xv[i - nblk]
        outT_ref[...] = jnp.maximum(
            _dot(W1as_ref[...], xb, ((1,), (0,)))
            + _dot(c2_ref[...], mask_for(base), ((1,), (0,))), 0.0)


def kernel(p, x, o, W1, b1, gamma, beta, W2, b2):
    del p
    n, d = x.shape
    nb = o.shape[0]
    nblk = n // _BLK
    xT = jnp.swapaxes(x, 0, 1)
    outT = pl.pallas_call(
        _body,
        grid=(2 * nblk,),
        in_specs=[
            pl.BlockSpec((1, nb), lambda i: (0, 0)),
            pl.BlockSpec(memory_space=pl.ANY),
            pl.BlockSpec((d, 2 * d), lambda i: (0, 0)),
            pl.BlockSpec((1, d), lambda i: (0, 0)),
            pl.BlockSpec((1, d), lambda i: (0, 0)),
            pl.BlockSpec((1, d), lambda i: (0, 0)),
            pl.BlockSpec((d, d), lambda i: (0, 0)),
            pl.BlockSpec((1, d), lambda i: (0, 0)),
        ],
        out_specs=pl.BlockSpec(
            (d, _BLK), lambda i: (0, jnp.maximum(i - n // _BLK, 0))),
        out_shape=jax.ShapeDtypeStruct((d, n), x.dtype),
        scratch_shapes=[
            pltpu.VMEM((n // _BLK, d, _BLK), jnp.float32),
            pltpu.VMEM((d, d), jnp.float32),
            pltpu.VMEM((d, nb), jnp.float32),
            pltpu.VMEM((d, d), jnp.float32),
            pltpu.VMEM((d, nb), jnp.float32),
            pltpu.SemaphoreType.DMA((2,)),
        ],
    )(o.reshape(1, nb), xT, W1, b1.reshape(1, d), gamma.reshape(1, d),
      beta.reshape(1, d), W2, b2.reshape(1, d))
    return jnp.swapaxes(outT, 0, 1)


# speedup vs baseline: 2.1024x; 1.1086x over previous
import jax
import jax.numpy as jnp
from jax import lax
from jax.experimental import pallas as pl
from jax.experimental.pallas import tpu as pltpu

_EPS = 1e-5
_BLK = 16384


def _dot(a, b, dims):
    return lax.dot_general(a, b, (dims, ((), ())),
                           preferred_element_type=jnp.float32)


def _body(o_ref, xT_hbm, W1_ref, b1_ref, gamma_ref, beta_ref, W2_ref, b2_ref,
          outT_ref, xv, G_ref, sums_ref, W1as_ref, c2_ref, sems):
    _, d, _ = xv.shape
    n = xv.shape[0] * _BLK
    nb = o_ref.shape[1]
    nblk = n // _BLK
    i = pl.program_id(0)
    nf = jnp.float32(n)

    of_row = o_ref[...].astype(jnp.float32)
    eye_b = (lax.broadcasted_iota(jnp.int32, (nb, nb), 0)
             == lax.broadcasted_iota(jnp.int32, (nb, nb), 1)
             ).astype(jnp.float32)
    o_col = _dot(eye_b, of_row, ((1,), (1,)))

    def mask_for(base):
        rf = (lax.broadcasted_iota(jnp.int32, (nb, _BLK), 1)
              + base).astype(jnp.float32)
        ltf = (rf < o_col).astype(jnp.float32)
        lt_prev = jnp.concatenate(
            [jnp.zeros((1, _BLK), jnp.float32), ltf[:-1, :]], axis=0)
        return ltf - lt_prev

    def copy_blk(j, sem):
        return pltpu.make_async_copy(
            xT_hbm.at[:, pl.ds(j * _BLK, _BLK)], xv.at[j], sem)

    @pl.when(i == 0)
    def _():
        G_ref[...] = jnp.zeros_like(G_ref)
        sums_ref[...] = jnp.zeros_like(sums_ref)
        copy_blk(0, sems.at[0]).start()

    @pl.when(i + 1 < nblk)
    def _():
        copy_blk(i + 1, sems.at[(i + 1) % 2]).start()

    @pl.when(i < nblk)
    def _():
        copy_blk(i, sems.at[i % 2]).wait()
        xb = xv[i]
        G_ref[...] += _dot(xb, xb, ((1,), (1,)))
        sums_ref[...] += _dot(xb, mask_for(i * _BLK), ((1,), (1,)))

    @pl.when(i == nblk)
    def _():
        prev_col = jnp.concatenate(
            [jnp.zeros((1, 1), jnp.float32), o_col[:-1, :]], axis=0)
        cnt_col = o_col - prev_col
        cnt_row = of_row - jnp.concatenate(
            [jnp.zeros((1, 1), jnp.float32), of_row[:, :-1]], axis=1)
        eye_d = (lax.broadcasted_iota(jnp.int32, (d, d), 0)
                 == lax.broadcasted_iota(jnp.int32, (d, d), 1)
                 ).astype(jnp.float32)
        vecs = jnp.concatenate(
            [b1_ref[...], gamma_ref[...], beta_ref[...], b2_ref[...]], axis=0)
        vecs_col = _dot(eye_d, vecs, ((1,), (1,)))
        b1_col = vecs_col[:, 0:1]
        gamma_col = vecs_col[:, 1:2]
        beta_col = vecs_col[:, 2:3]
        b2_col = vecs_col[:, 3:4]

        sums_T = sums_ref[...]
        m_T = sums_T * (1.0 / cnt_row)
        h_T = jnp.maximum(
            _dot(W2_ref[...], m_T, ((1,), (0,))) + b2_col, 0.0)
        W1 = W1_ref[...]
        W1a = W1[:, :d]
        W1b = W1[:, d:]
        c_T = _dot(W1b, h_T, ((1,), (0,))) + b1_col

        seg_t_T = _dot(W1a, sums_T, ((1,), (0,)))
        sum_y = (jnp.sum(seg_t_T, axis=1, keepdims=True)
                 + jnp.sum(c_T * cnt_row, axis=1, keepdims=True))
        W1aG = _dot(W1a, G_ref[...], ((1,), (0,)))
        sumsq_t = jnp.sum(W1a * W1aG, axis=1, keepdims=True)
        sumsq_y = (sumsq_t
                   + 2.0 * jnp.sum(c_T * seg_t_T, axis=1, keepdims=True)
                   + jnp.sum(c_T * c_T * cnt_row, axis=1, keepdims=True))
        mean = sum_y / nf
        var = sumsq_y / nf - mean * mean
        a_col = gamma_col * lax.rsqrt(var + _EPS)
        bsh_col = beta_col - mean * a_col
        W1as_ref[...] = W1a * a_col
        c2_ref[...] = c_T * a_col + bsh_col

    @pl.when(i >= nblk)
    def _():
        base = (i - nblk) * _BLK
        xb = xv[i - nblk]
        outT_ref[...] = jnp.maximum(
            _dot(W1as_ref[...], xb, ((1,), (0,)))
            + _dot(c2_ref[...], mask_for(base), ((1,), (0,))), 0.0)


def kernel(p, x, o, W1, b1, gamma, beta, W2, b2):
    del p
    n, d = x.shape
    nb = o.shape[0]
    nblk = n // _BLK
    xT = jnp.swapaxes(x, 0, 1)
    outT = pl.pallas_call(
        _body,
        grid=(2 * nblk,),
        in_specs=[
            pl.BlockSpec((1, nb), lambda i: (0, 0)),
            pl.BlockSpec(memory_space=pl.ANY),
            pl.BlockSpec((d, 2 * d), lambda i: (0, 0)),
            pl.BlockSpec((1, d), lambda i: (0, 0)),
            pl.BlockSpec((1, d), lambda i: (0, 0)),
            pl.BlockSpec((1, d), lambda i: (0, 0)),
            pl.BlockSpec((d, d), lambda i: (0, 0)),
            pl.BlockSpec((1, d), lambda i: (0, 0)),
        ],
        out_specs=pl.BlockSpec(
            (d, _BLK), lambda i: (0, jnp.maximum(i - n // _BLK, 0))),
        out_shape=jax.ShapeDtypeStruct((d, n), x.dtype),
        scratch_shapes=[
            pltpu.VMEM((n // _BLK, d, _BLK), jnp.float32),
            pltpu.VMEM((d, d), jnp.float32),
            pltpu.VMEM((d, nb), jnp.float32),
            pltpu.VMEM((d, d), jnp.float32),
            pltpu.VMEM((d, nb), jnp.float32),
            pltpu.SemaphoreType.DMA((2,)),
        ],
    )(o.reshape(1, nb), xT, W1, b1.reshape(1, d), gamma.reshape(1, d),
      beta.reshape(1, d), W2, b2.reshape(1, d))
    return jnp.swapaxes(outT, 0, 1)


# speedup vs baseline: 2.1215x; 1.0091x over previous
import jax
import jax.numpy as jnp
from jax import lax
from jax.experimental import pallas as pl
from jax.experimental.pallas import tpu as pltpu

_EPS = 1e-5
_BLK = 16384


def _dot(a, b, dims):
    return lax.dot_general(a, b, (dims, ((), ())),
                           preferred_element_type=jnp.float32)


def _body(o_ref, xT_hbm, W1_ref, b1_ref, gamma_ref, beta_ref, W2_ref, b2_ref,
          outT_ref, xv, maskv, G_ref, sums_ref, W1as_ref, c2_ref, sems):
    _, d, _ = xv.shape
    n = xv.shape[0] * _BLK
    nb = o_ref.shape[1]
    nblk = n // _BLK
    i = pl.program_id(0)
    nf = jnp.float32(n)

    of_row = o_ref[...].astype(jnp.float32)
    eye_b = (lax.broadcasted_iota(jnp.int32, (nb, nb), 0)
             == lax.broadcasted_iota(jnp.int32, (nb, nb), 1)
             ).astype(jnp.float32)
    o_col = _dot(eye_b, of_row, ((1,), (1,)))

    def mask_for(base):
        rf = (lax.broadcasted_iota(jnp.int32, (nb, _BLK), 1)
              + base).astype(jnp.float32)
        ltf = (rf < o_col).astype(jnp.float32)
        lt_prev = jnp.concatenate(
            [jnp.zeros((1, _BLK), jnp.float32), ltf[:-1, :]], axis=0)
        return ltf - lt_prev

    def copy_blk(j, sem):
        return pltpu.make_async_copy(
            xT_hbm.at[:, pl.ds(j * _BLK, _BLK)], xv.at[j], sem)

    @pl.when(i == 0)
    def _():
        G_ref[...] = jnp.zeros_like(G_ref)
        sums_ref[...] = jnp.zeros_like(sums_ref)
        copy_blk(0, sems.at[0]).start()

    @pl.when(i + 1 < nblk)
    def _():
        copy_blk(i + 1, sems.at[(i + 1) % 2]).start()

    @pl.when(i < nblk)
    def _():
        copy_blk(i, sems.at[i % 2]).wait()
        xb = xv[i]
        mb = mask_for(i * _BLK)
        maskv[i] = mb
        G_ref[...] += _dot(xb, xb, ((1,), (1,)))
        sums_ref[...] += _dot(xb, mb, ((1,), (1,)))

    @pl.when(i == nblk)
    def _():
        prev_col = jnp.concatenate(
            [jnp.zeros((1, 1), jnp.float32), o_col[:-1, :]], axis=0)
        cnt_col = o_col - prev_col
        cnt_row = of_row - jnp.concatenate(
            [jnp.zeros((1, 1), jnp.float32), of_row[:, :-1]], axis=1)
        eye_d = (lax.broadcasted_iota(jnp.int32, (d, d), 0)
                 == lax.broadcasted_iota(jnp.int32, (d, d), 1)
                 ).astype(jnp.float32)
        vecs = jnp.concatenate(
            [b1_ref[...], gamma_ref[...], beta_ref[...], b2_ref[...]], axis=0)
        vecs_col = _dot(eye_d, vecs, ((1,), (1,)))
        b1_col = vecs_col[:, 0:1]
        gamma_col = vecs_col[:, 1:2]
        beta_col = vecs_col[:, 2:3]
        b2_col = vecs_col[:, 3:4]

        sums_T = sums_ref[...]
        m_T = sums_T * (1.0 / cnt_row)
        h_T = jnp.maximum(
            _dot(W2_ref[...], m_T, ((1,), (0,))) + b2_col, 0.0)
        W1 = W1_ref[...]
        W1a = W1[:, :d]
        W1b = W1[:, d:]
        c_T = _dot(W1b, h_T, ((1,), (0,))) + b1_col

        seg_t_T = _dot(W1a, sums_T, ((1,), (0,)))
        sum_y = (jnp.sum(seg_t_T, axis=1, keepdims=True)
                 + jnp.sum(c_T * cnt_row, axis=1, keepdims=True))
        W1aG = _dot(W1a, G_ref[...], ((1,), (0,)))
        sumsq_t = jnp.sum(W1a * W1aG, axis=1, keepdims=True)
        sumsq_y = (sumsq_t
                   + 2.0 * jnp.sum(c_T * seg_t_T, axis=1, keepdims=True)
                   + jnp.sum(c_T * c_T * cnt_row, axis=1, keepdims=True))
        mean = sum_y / nf
        var = sumsq_y / nf - mean * mean
        a_col = gamma_col * lax.rsqrt(var + _EPS)
        bsh_col = beta_col - mean * a_col
        W1as_ref[...] = W1a * a_col
        c2_ref[...] = c_T * a_col + bsh_col

    @pl.when(i >= nblk)
    def _():
        xb = xv[i - nblk]
        outT_ref[...] = jnp.maximum(
            _dot(W1as_ref[...], xb, ((1,), (0,)))
            + _dot(c2_ref[...], maskv[i - nblk], ((1,), (0,))), 0.0)


def kernel(p, x, o, W1, b1, gamma, beta, W2, b2):
    del p
    n, d = x.shape
    nb = o.shape[0]
    nblk = n // _BLK
    xT = jnp.swapaxes(x, 0, 1)
    outT = pl.pallas_call(
        _body,
        grid=(2 * nblk,),
        in_specs=[
            pl.BlockSpec((1, nb), lambda i: (0, 0)),
            pl.BlockSpec(memory_space=pl.ANY),
            pl.BlockSpec((d, 2 * d), lambda i: (0, 0)),
            pl.BlockSpec((1, d), lambda i: (0, 0)),
            pl.BlockSpec((1, d), lambda i: (0, 0)),
            pl.BlockSpec((1, d), lambda i: (0, 0)),
            pl.BlockSpec((d, d), lambda i: (0, 0)),
            pl.BlockSpec((1, d), lambda i: (0, 0)),
        ],
        out_specs=pl.BlockSpec(
            (d, _BLK), lambda i: (0, jnp.maximum(i - n // _BLK, 0))),
        out_shape=jax.ShapeDtypeStruct((d, n), x.dtype),
        scratch_shapes=[
            pltpu.VMEM((n // _BLK, d, _BLK), jnp.float32),
            pltpu.VMEM((n // _BLK, nb, _BLK), jnp.float32),
            pltpu.VMEM((d, d), jnp.float32),
            pltpu.VMEM((d, nb), jnp.float32),
            pltpu.VMEM((d, d), jnp.float32),
            pltpu.VMEM((d, nb), jnp.float32),
            pltpu.SemaphoreType.DMA((2,)),
        ],
    )(o.reshape(1, nb), xT, W1, b1.reshape(1, d), gamma.reshape(1, d),
      beta.reshape(1, d), W2, b2.reshape(1, d))
    return jnp.swapaxes(outT, 0, 1)
